# Initial kernel scaffold; baseline (speedup 1.0000x reference)
import jax
import jax.numpy as jnp
from jax.experimental import pallas as pl


def kernel(x_ingredient, x_taste, edge_src, edge_dst, W_ing, b_ing, W_taste, b_taste, att_src, att_dst, Wk, bk, q, gamma, beta):
    raise NotImplementedError("write your pallas kernel here")


# speedup vs baseline: 1.3192x; 1.3192x over previous
import jax
import jax.numpy as jnp
from jax.experimental import pallas as pl
from jax.experimental.pallas import tpu as pltpu


def _finalize_body(acc_ref, d_ref, xt_ref, w_ref, b_ref, g_ref, bb_ref, out_ref):
    acc = acc_ref[...]
    h = jnp.dot(acc, w_ref[...], preferred_element_type=jnp.float32)
    has = (d_ref[...] > 0.0).astype(jnp.float32)
    h = h + has[0][:, None] * b_ref[...][0][None, :]
    h = jnp.maximum(h, 0.0)
    t = h + xt_ref[...]
    mean = jnp.mean(t, axis=0, keepdims=True)
    var = jnp.mean((t - mean) ** 2, axis=0, keepdims=True)
    y = (t - mean) / jnp.sqrt(var + 1e-5) * g_ref[...][0][None, :] + bb_ref[...][0][None, :]
    out_ref[...] = jnp.maximum(y, 0.0)


def kernel(x_ingredient, x_taste, edge_src, edge_dst, W_ing, b_ing, W_taste,
           b_taste, att_src, att_dst, Wk, bk, q, gamma, beta):
    N_ing, D = x_ingredient.shape
    N_taste = x_taste.shape[0]

    u = W_ing @ att_src
    v = W_taste @ att_dst
    a_src = x_ingredient @ u + b_ing @ att_src
    a_dst = x_taste @ v + b_taste @ att_dst
    alpha = a_src[edge_src] + a_dst[edge_dst]
    alpha = jax.nn.leaky_relu(alpha, 0.2)
    m = jnp.max(alpha)
    ex = jnp.exp(alpha - m)
    denom = jax.ops.segment_sum(ex, edge_dst, num_segments=N_taste)
    w = ex / jnp.maximum(denom[edge_dst], 1e-16)
    agg = jax.ops.segment_sum(x_ingredient[edge_src] * w[:, None], edge_dst,
                              num_segments=N_taste)

    out_t = pl.pallas_call(
        _finalize_body,
        out_shape=jax.ShapeDtypeStruct((N_taste, D), jnp.float32),
    )(agg, denom[None, :], x_taste,
      W_ing, b_ing[None, :], gamma[None, :], beta[None, :])
    return (x_ingredient, out_t)


# speedup vs baseline: 24.0858x; 18.2576x over previous
import functools

import jax
import jax.numpy as jnp
from jax import lax
from jax.experimental import pallas as pl
from jax.experimental.pallas import tpu as pltpu
from jax.experimental.pallas import tpu_sc as plsc

NC = 2
NS = 16
NW = NC * NS
L = 16
C1 = 256
C2 = 128
DA = 144
BR = 2048

_MESH = plsc.VectorSubcoreMesh(core_axis_name="c", subcore_axis_name="s")


def _matvec_body(nvalid, br, x_ref, w_ref, att_ref, b_ref, out_ref, max_ref):
    i = pl.program_id(0)
    u = jnp.dot(w_ref[...], att_ref[...][0][:, None],
                preferred_element_type=jnp.float32)
    c = jnp.sum(b_ref[...] * att_ref[...])
    s = jnp.dot(x_ref[...], u, preferred_element_type=jnp.float32) + c
    s2 = s.reshape(br // 128, 128)
    row = lax.broadcasted_iota(jnp.int32, (br // 128, 128), 0)
    col = lax.broadcasted_iota(jnp.int32, (br // 128, 128), 1)
    valid = (i * br + row * 128 + col) < nvalid
    out_ref[...] = jnp.where(valid, s2, 0.0)

    @pl.when(i == 0)
    def _():
        max_ref[0, 0] = -1e30

    max_ref[0, 0] = jnp.maximum(max_ref[0, 0],
                                jnp.max(jnp.where(valid, s2, -1e30)))


def _matvec(x, w, att, b, nvalid, br):
    n, d = x.shape
    nblk = (n + br - 1) // br
    out, mx = pl.pallas_call(
        functools.partial(_matvec_body, nvalid, br),
        grid=(nblk,),
        in_specs=[
            pl.BlockSpec((br, d), lambda i: (i, 0)),
            pl.BlockSpec((d, d), lambda i: (0, 0)),
            pl.BlockSpec((1, d), lambda i: (0, 0)),
            pl.BlockSpec((1, d), lambda i: (0, 0)),
        ],
        out_specs=[
            pl.BlockSpec((br // 128, 128), lambda i: (i, 0)),
            pl.BlockSpec(memory_space=pltpu.SMEM),
        ],
        out_shape=[
            jax.ShapeDtypeStruct((nblk * br // 128, 128), jnp.float32),
            jax.ShapeDtypeStruct((1, 1), jnp.float32),
        ],
    )(x, w, att[None, :], b[None, :])
    return out.reshape(-1), mx[0, 0]


def _sc_pass1(a_src, a_dst, src_pad, dst_pad, m16, k1):
    asp = a_src.shape[0]
    ntp = a_dst.shape[0]
    ep = src_pad.shape[0]

    @functools.partial(
        pl.kernel,
        out_type=jax.ShapeDtypeStruct((ep,), jnp.float32),
        mesh=_MESH,
        compiler_params=pltpu.CompilerParams(needs_layout_passes=False),
        scratch_types=[
            pltpu.VMEM((asp,), jnp.float32),
            pltpu.VMEM((ntp,), jnp.float32),
            pltpu.VMEM((C1,), jnp.int32),
            pltpu.VMEM((C1,), jnp.int32),
            pltpu.VMEM((C1,), jnp.float32),
            pltpu.VMEM((L,), jnp.float32),
        ],
    )
    def pass1(asrc_hbm, adst_hbm, src_hbm, dst_hbm, m_hbm, ex_hbm,
              asrc_t, adst_t, sidx, didx, ex_v, m_v):
        ci = lax.axis_index("c")
        si = lax.axis_index("s")
        wid = ci * NS + si
        pltpu.sync_copy(asrc_hbm, asrc_t)
        pltpu.sync_copy(adst_hbm, adst_t)
        pltpu.sync_copy(m_hbm, m_v)
        mvec = m_v[...]

        def body(k, _):
            base = wid * k1 * C1 + k * C1
            pltpu.sync_copy(src_hbm.at[pl.ds(base, C1)], sidx)
            pltpu.sync_copy(dst_hbm.at[pl.ds(base, C1)], didx)
            for i in range(C1 // L):
                sl = pl.ds(i * L, L)
                al = (plsc.load_gather(asrc_t, [sidx[sl]])
                      + plsc.load_gather(adst_t, [didx[sl]]))
                al = jnp.maximum(al, 0.0) + 0.2 * jnp.minimum(al, 0.0)
                ex_v[sl] = jnp.exp(al - mvec)
            pltpu.sync_copy(ex_v, ex_hbm.at[pl.ds(base, C1)])
            return 0
        lax.fori_loop(0, k1, body, 0)

    return pass1(a_src, a_dst, src_pad, dst_pad, m16)


def _sc_pass2(x, src_pad, dst_pad, ex, ntp, k2):
    d = x.shape[1]
    rpt = ntp // NS

    @functools.partial(
        pl.kernel,
        out_type=jax.ShapeDtypeStruct((NC, ntp, DA), jnp.float32),
        mesh=_MESH,
        compiler_params=pltpu.CompilerParams(needs_layout_passes=False,
                                             use_tc_tiling_on_sc=False),
        scratch_types=[
            pltpu.VMEM((C2,), jnp.int32),
            pltpu.VMEM((C2,), jnp.int32),
            pltpu.VMEM((C2,), jnp.float32),
            pltpu.VMEM((C2, 128), jnp.float32),
            pltpu.VMEM((C2, DA), jnp.float32),
            pltpu.VMEM_SHARED((ntp, DA), jnp.float32),
            pltpu.SemaphoreType.DMA,
            pltpu.SemaphoreType.DMA,
        ],
    )
    def pass2(x_hbm, src_hbm, dst_hbm, ex_hbm, acc_hbm,
              sidx, didx, ex_v, xg, rows, acc_sh, sem, sem2):
        ci = lax.axis_index("c")
        si = lax.axis_index("s")
        wid = ci * NS + si

        for j in range(C2):
            for jj in range(DA // L):
                rows[j, pl.ds(jj * L, L)] = jnp.zeros((L,), jnp.float32)

        def zcopy(r, _):
            pltpu.sync_copy(rows, acc_sh.at[pl.ds(si * rpt + r * C2, C2), :])
            return 0
        lax.fori_loop(0, rpt // C2, zcopy, 0)
        plsc.subcore_barrier()

        onehot = jnp.where(lax.iota(jnp.int32, L) == 0, 1.0, 0.0)
        ones = jnp.ones((L,), jnp.float32)

        def body(k, _):
            base = wid * k2 * C2 + k * C2
            pltpu.sync_copy(src_hbm.at[pl.ds(base, C2)], sidx)
            pltpu.sync_copy(dst_hbm.at[pl.ds(base, C2)], didx)
            pltpu.sync_copy(ex_hbm.at[pl.ds(base, C2)], ex_v)
            pltpu.async_copy(x_hbm.at[sidx], xg, sem).wait()
            for g in range(C2 // L):
                ev = ex_v[pl.ds(g * L, L)]
                for lane in range(L):
                    r = g * L + lane
                    exb = ev[lane] * ones
                    for j in range(d // L):
                        sl = pl.ds(j * L, L)
                        rows[r, sl] = xg[r, sl] * exb
                    rows[r, pl.ds(d, L)] = exb * onehot
            pltpu.async_copy(rows, acc_sh.at[didx], sem2, add=True).wait()
            return 0
        lax.fori_loop(0, k2, body, 0)
        plsc.subcore_barrier()
        pltpu.sync_copy(acc_sh.at[pl.ds(si * rpt, rpt), :],
                        acc_hbm.at[ci, pl.ds(si * rpt, rpt), :])

    return pass2(x, src_pad, dst_pad, ex)


def _final_body(nt, d, ap_ref, xt_ref, w_ref, b_ref, g_ref, bb_ref, out_ref):
    xagg = ap_ref[0, :nt, :d] + ap_ref[1, :nt, :d]
    den = ap_ref[0, :nt, d:d + 1] + ap_ref[1, :nt, d:d + 1]
    xn = xagg / jnp.maximum(den, 1e-16)
    has = (den > 0.0).astype(jnp.float32)
    h = jnp.dot(xn, w_ref[...], preferred_element_type=jnp.float32)
    h = h + has * b_ref[...][0][None, :]
    h = jnp.maximum(h, 0.0)
    t = h + xt_ref[...]
    mean = jnp.mean(t, axis=0, keepdims=True)
    var = jnp.mean((t - mean) ** 2, axis=0, keepdims=True)
    y = (t - mean) / jnp.sqrt(var + 1e-5) * g_ref[...][0][None, :] \
        + bb_ref[...][0][None, :]
    out_ref[...] = jnp.maximum(y, 0.0)


def kernel(x_ingredient, x_taste, edge_src, edge_dst, W_ing, b_ing, W_taste,
           b_taste, att_src, att_dst, Wk, bk, q, gamma, beta):
    n_ing, d = x_ingredient.shape
    nt = x_taste.shape[0]
    e = edge_src.shape[0]
    ntp = ((nt + 1 + NS * L - 1) // (NS * L)) * (NS * L)
    k1 = (e + NW * C1 - 1) // (NW * C1)
    ep = NW * C1 * k1
    k2 = ep // (NW * C2)

    a_src, ms = _matvec(x_ingredient, W_ing, att_src, b_ing, n_ing, BR)
    a_dst, md = _matvec(x_taste, W_taste, att_dst, b_taste, nt, BR)
    a_dst = a_dst[:ntp]
    t = ms + md
    m = jnp.maximum(t, 0.0) + 0.2 * jnp.minimum(t, 0.0)
    m16 = jnp.full((L,), 1.0, jnp.float32) * m

    src_pad = jnp.concatenate([edge_src, jnp.zeros((ep - e,), jnp.int32)])
    dst_pad = jnp.concatenate([edge_dst, jnp.full((ep - e,), nt, jnp.int32)])

    ex = _sc_pass1(a_src, a_dst, src_pad, dst_pad, m16, k1)
    acc_parts = _sc_pass2(x_ingredient, src_pad, dst_pad, ex, ntp, k2)

    out_t = pl.pallas_call(
        functools.partial(_final_body, nt, d),
        out_shape=jax.ShapeDtypeStruct((nt, d), jnp.float32),
    )(acc_parts, x_taste, W_ing, b_ing[None, :], gamma[None, :], beta[None, :])
    return (x_ingredient, out_t)


# speedup vs baseline: 29.7453x; 1.2350x over previous
import functools

import jax
import jax.numpy as jnp
from jax import lax
from jax.experimental import pallas as pl
from jax.experimental.pallas import tpu as pltpu
from jax.experimental.pallas import tpu_sc as plsc

NC = 2
NS = 16
NW = NC * NS
L = 16
C1 = 256
C2 = 64
DA = 144
BR = 2048

_MESH = plsc.VectorSubcoreMesh(core_axis_name="c", subcore_axis_name="s")


def _matvec_body(nvalid, br, x_ref, w_ref, att_ref, b_ref, out_ref, max_ref):
    i = pl.program_id(0)
    u = jnp.dot(w_ref[...], att_ref[...][0][:, None],
                preferred_element_type=jnp.float32)
    c = jnp.sum(b_ref[...] * att_ref[...])
    s = jnp.dot(x_ref[...], u, preferred_element_type=jnp.float32) + c
    s2 = s.reshape(br // 128, 128)
    row = lax.broadcasted_iota(jnp.int32, (br // 128, 128), 0)
    col = lax.broadcasted_iota(jnp.int32, (br // 128, 128), 1)
    valid = (i * br + row * 128 + col) < nvalid
    out_ref[...] = jnp.where(valid, s2, 0.0)

    @pl.when(i == 0)
    def _():
        max_ref[0, 0] = -1e30

    max_ref[0, 0] = jnp.maximum(max_ref[0, 0],
                                jnp.max(jnp.where(valid, s2, -1e30)))


def _matvec(x, w, att, b, nvalid, br):
    n, d = x.shape
    nblk = (n + br - 1) // br
    out, mx = pl.pallas_call(
        functools.partial(_matvec_body, nvalid, br),
        grid=(nblk,),
        in_specs=[
            pl.BlockSpec((br, d), lambda i: (i, 0)),
            pl.BlockSpec((d, d), lambda i: (0, 0)),
            pl.BlockSpec((1, d), lambda i: (0, 0)),
            pl.BlockSpec((1, d), lambda i: (0, 0)),
        ],
        out_specs=[
            pl.BlockSpec((br // 128, 128), lambda i: (i, 0)),
            pl.BlockSpec(memory_space=pltpu.SMEM),
        ],
        out_shape=[
            jax.ShapeDtypeStruct((nblk * br // 128, 128), jnp.float32),
            jax.ShapeDtypeStruct((1, 1), jnp.float32),
        ],
    )(x, w, att[None, :], b[None, :])
    return out.reshape(-1), mx[0, 0]


def _sc_pass1(a_src, a_dst, src_pad, dst_pad, m16, k1):
    asp = a_src.shape[0]
    ntp = a_dst.shape[0]
    ep = src_pad.shape[0]

    @functools.partial(
        pl.kernel,
        out_type=jax.ShapeDtypeStruct((ep,), jnp.float32),
        mesh=_MESH,
        compiler_params=pltpu.CompilerParams(needs_layout_passes=False),
        scratch_types=[
            pltpu.VMEM((asp,), jnp.float32),
            pltpu.VMEM((ntp,), jnp.float32),
            pltpu.VMEM((C1,), jnp.int32),
            pltpu.VMEM((C1,), jnp.int32),
            pltpu.VMEM((C1,), jnp.float32),
            pltpu.VMEM((L,), jnp.float32),
        ],
    )
    def pass1(asrc_hbm, adst_hbm, src_hbm, dst_hbm, m_hbm, ex_hbm,
              asrc_t, adst_t, sidx, didx, ex_v, m_v):
        ci = lax.axis_index("c")
        si = lax.axis_index("s")
        wid = ci * NS + si
        pltpu.sync_copy(asrc_hbm, asrc_t)
        pltpu.sync_copy(adst_hbm, adst_t)
        pltpu.sync_copy(m_hbm, m_v)
        mvec = m_v[...]

        def body(k, _):
            base = wid * k1 * C1 + k * C1
            pltpu.sync_copy(src_hbm.at[pl.ds(base, C1)], sidx)
            pltpu.sync_copy(dst_hbm.at[pl.ds(base, C1)], didx)
            for i in range(C1 // L):
                sl = pl.ds(i * L, L)
                al = (plsc.load_gather(asrc_t, [sidx[sl]])
                      + plsc.load_gather(adst_t, [didx[sl]]))
                al = jnp.maximum(al, 0.0) + 0.2 * jnp.minimum(al, 0.0)
                ex_v[sl] = jnp.exp(al - mvec)
            pltpu.sync_copy(ex_v, ex_hbm.at[pl.ds(base, C1)])
            return 0
        lax.fori_loop(0, k1, body, 0)

    return pass1(a_src, a_dst, src_pad, dst_pad, m16)


def _sc_pass2(x, src_pad, dst_pad, ex, ntp, k2):
    d = x.shape[1]
    rpt = ntp // NS

    @functools.partial(
        pl.kernel,
        out_type=jax.ShapeDtypeStruct((NC, ntp, DA), jnp.float32),
        mesh=_MESH,
        compiler_params=pltpu.CompilerParams(needs_layout_passes=False,
                                             use_tc_tiling_on_sc=False),
        scratch_types=[
            pltpu.VMEM((2, C2), jnp.int32),
            pltpu.VMEM((2, C2), jnp.int32),
            pltpu.VMEM((2, C2), jnp.float32),
            pltpu.VMEM((2, C2, 128), jnp.float32),
            pltpu.VMEM((2, C2, DA), jnp.float32),
            pltpu.VMEM_SHARED((ntp, DA), jnp.float32),
            pltpu.SemaphoreType.DMA,
            pltpu.SemaphoreType.DMA,
            pltpu.SemaphoreType.DMA,
            pltpu.SemaphoreType.DMA,
        ],
    )
    def pass2(x_hbm, src_hbm, dst_hbm, ex_hbm, acc_hbm,
              sidx, didx, ex_v, xg, rows, acc_sh, g0, g1, s0, s1):
        ci = lax.axis_index("c")
        si = lax.axis_index("s")
        wid = ci * NS + si
        tbase = wid * k2 * C2
        gsem = (g0, g1)
        ssem = (s0, s1)

        for j in range(C2):
            for jj in range(DA // L):
                rows[0, j, pl.ds(jj * L, L)] = jnp.zeros((L,), jnp.float32)

        def zcopy(r, _):
            pltpu.sync_copy(rows.at[0],
                            acc_sh.at[pl.ds(si * rpt + r * C2, C2), :])
            return 0
        lax.fori_loop(0, rpt // C2, zcopy, 0)
        plsc.subcore_barrier()

        onehot = jnp.where(lax.iota(jnp.int32, L) == 0, 1.0, 0.0)
        ones = jnp.ones((L,), jnp.float32)

        def start_gather(p, base):
            pltpu.sync_copy(src_hbm.at[pl.ds(base, C2)], sidx.at[p])
            pltpu.sync_copy(ex_hbm.at[pl.ds(base, C2)], ex_v.at[p])
            pltpu.async_copy(x_hbm.at[sidx.at[p]], xg.at[p], gsem[p])

        def leg(p, t, base):
            pltpu.make_async_copy(x_hbm.at[sidx.at[p]], xg.at[p],
                                  gsem[p]).wait()

            @pl.when(t > 0)
            def _():
                pltpu.make_async_copy(rows.at[p], acc_sh.at[didx.at[p]],
                                      ssem[p]).wait()
            pltpu.sync_copy(dst_hbm.at[pl.ds(base, C2)], didx.at[p])
            for g in range(C2 // L):
                ev = ex_v[p, pl.ds(g * L, L)]
                for lane in range(L):
                    r = g * L + lane
                    exb = ev[lane] * ones
                    for j in range(d // L):
                        sl = pl.ds(j * L, L)
                        rows[p, r, sl] = xg[p, r, sl] * exb
                    rows[p, r, pl.ds(d, L)] = exb * onehot
            pltpu.async_copy(rows.at[p], acc_sh.at[didx.at[p]], ssem[p],
                             add=True)

        start_gather(0, tbase)

        def body(t, _):
            a = tbase + (2 * t) * C2
            start_gather(1, a + C2)
            leg(0, t, a)

            @pl.when(t < k2 // 2 - 1)
            def _():
                start_gather(0, a + 2 * C2)
            leg(1, t, a + C2)
            return 0
        lax.fori_loop(0, k2 // 2, body, 0)
        pltpu.make_async_copy(rows.at[0], acc_sh.at[didx.at[0]], s0).wait()
        pltpu.make_async_copy(rows.at[1], acc_sh.at[didx.at[1]], s1).wait()
        plsc.subcore_barrier()
        pltpu.sync_copy(acc_sh.at[pl.ds(si * rpt, rpt), :],
                        acc_hbm.at[ci, pl.ds(si * rpt, rpt), :])

    return pass2(x, src_pad, dst_pad, ex)


def _final_body(nt, d, ap_ref, xt_ref, w_ref, b_ref, g_ref, bb_ref, out_ref):
    xagg = ap_ref[0, :nt, :d] + ap_ref[1, :nt, :d]
    den = ap_ref[0, :nt, d:d + 1] + ap_ref[1, :nt, d:d + 1]
    xn = xagg / jnp.maximum(den, 1e-16)
    has = (den > 0.0).astype(jnp.float32)
    h = jnp.dot(xn, w_ref[...], preferred_element_type=jnp.float32)
    h = h + has * b_ref[...][0][None, :]
    h = jnp.maximum(h, 0.0)
    t = h + xt_ref[...]
    mean = jnp.mean(t, axis=0, keepdims=True)
    var = jnp.mean((t - mean) ** 2, axis=0, keepdims=True)
    y = (t - mean) / jnp.sqrt(var + 1e-5) * g_ref[...][0][None, :] \
        + bb_ref[...][0][None, :]
    out_ref[...] = jnp.maximum(y, 0.0)


def kernel(x_ingredient, x_taste, edge_src, edge_dst, W_ing, b_ing, W_taste,
           b_taste, att_src, att_dst, Wk, bk, q, gamma, beta):
    n_ing, d = x_ingredient.shape
    nt = x_taste.shape[0]
    e = edge_src.shape[0]
    ntp = ((nt + 1 + NS * L - 1) // (NS * L)) * (NS * L)
    k1 = (e + NW * C1 - 1) // (NW * C1)
    ep = NW * C1 * k1
    k2 = ep // (NW * C2)

    a_src, ms = _matvec(x_ingredient, W_ing, att_src, b_ing, n_ing, BR)
    a_dst, md = _matvec(x_taste, W_taste, att_dst, b_taste, nt, BR)
    a_dst = a_dst[:ntp]
    t = ms + md
    m = jnp.maximum(t, 0.0) + 0.2 * jnp.minimum(t, 0.0)
    m16 = jnp.full((L,), 1.0, jnp.float32) * m

    src_pad = jnp.concatenate([edge_src, jnp.zeros((ep - e,), jnp.int32)])
    dst_pad = jnp.concatenate([edge_dst, jnp.full((ep - e,), nt, jnp.int32)])

    ex = _sc_pass1(a_src, a_dst, src_pad, dst_pad, m16, k1)
    acc_parts = _sc_pass2(x_ingredient, src_pad, dst_pad, ex, ntp, k2)

    out_t = pl.pallas_call(
        functools.partial(_final_body, nt, d),
        out_shape=jax.ShapeDtypeStruct((nt, d), jnp.float32),
    )(acc_parts, x_taste, W_ing, b_ing[None, :], gamma[None, :], beta[None, :])
    return (x_ingredient, out_t)


# speedup vs baseline: 30.7188x; 1.0327x over previous
import functools

import jax
import jax.numpy as jnp
from jax import lax
from jax.experimental import pallas as pl
from jax.experimental.pallas import tpu as pltpu
from jax.experimental.pallas import tpu_sc as plsc

NC = 2
NS = 16
NW = NC * NS
L = 16
C1 = 256
C2 = 64
DA = 144
BR = 2048

_MESH = plsc.VectorSubcoreMesh(core_axis_name="c", subcore_axis_name="s")


def _matvec_body(nvalid, br, x_ref, w_ref, att_ref, b_ref, out_ref, max_ref):
    i = pl.program_id(0)
    u = jnp.dot(w_ref[...], att_ref[...][0][:, None],
                preferred_element_type=jnp.float32)
    c = jnp.sum(b_ref[...] * att_ref[...])
    s = jnp.dot(x_ref[...], u, preferred_element_type=jnp.float32) + c
    s2 = s.reshape(br // 128, 128)
    row = lax.broadcasted_iota(jnp.int32, (br // 128, 128), 0)
    col = lax.broadcasted_iota(jnp.int32, (br // 128, 128), 1)
    valid = (i * br + row * 128 + col) < nvalid
    out_ref[...] = jnp.where(valid, s2, 0.0)

    @pl.when(i == 0)
    def _():
        max_ref[0, 0] = -1e30

    max_ref[0, 0] = jnp.maximum(max_ref[0, 0],
                                jnp.max(jnp.where(valid, s2, -1e30)))


def _matvec(x, w, att, b, nvalid, br):
    n, d = x.shape
    nblk = (n + br - 1) // br
    out, mx = pl.pallas_call(
        functools.partial(_matvec_body, nvalid, br),
        grid=(nblk,),
        in_specs=[
            pl.BlockSpec((br, d), lambda i: (i, 0)),
            pl.BlockSpec((d, d), lambda i: (0, 0)),
            pl.BlockSpec((1, d), lambda i: (0, 0)),
            pl.BlockSpec((1, d), lambda i: (0, 0)),
        ],
        out_specs=[
            pl.BlockSpec((br // 128, 128), lambda i: (i, 0)),
            pl.BlockSpec(memory_space=pltpu.SMEM),
        ],
        out_shape=[
            jax.ShapeDtypeStruct((nblk * br // 128, 128), jnp.float32),
            jax.ShapeDtypeStruct((1, 1), jnp.float32),
        ],
    )(x, w, att[None, :], b[None, :])
    return out.reshape(-1), mx[0, 0]


def _sc_pass1(a_src, a_dst, src_pad, dst_pad, m16, k1):
    asp = a_src.shape[0]
    ntp = a_dst.shape[0]
    ep = src_pad.shape[0]

    @functools.partial(
        pl.kernel,
        out_type=jax.ShapeDtypeStruct((ep,), jnp.float32),
        mesh=_MESH,
        compiler_params=pltpu.CompilerParams(needs_layout_passes=False),
        scratch_types=[
            pltpu.VMEM((asp,), jnp.float32),
            pltpu.VMEM((ntp,), jnp.float32),
            pltpu.VMEM((C1,), jnp.int32),
            pltpu.VMEM((C1,), jnp.int32),
            pltpu.VMEM((C1,), jnp.float32),
            pltpu.VMEM((L,), jnp.float32),
        ],
    )
    def pass1(asrc_hbm, adst_hbm, src_hbm, dst_hbm, m_hbm, ex_hbm,
              asrc_t, adst_t, sidx, didx, ex_v, m_v):
        ci = lax.axis_index("c")
        si = lax.axis_index("s")
        wid = ci * NS + si
        pltpu.sync_copy(asrc_hbm, asrc_t)
        pltpu.sync_copy(adst_hbm, adst_t)
        pltpu.sync_copy(m_hbm, m_v)
        mvec = m_v[...]

        def body(k, _):
            base = wid * k1 * C1 + k * C1
            pltpu.sync_copy(src_hbm.at[pl.ds(base, C1)], sidx)
            pltpu.sync_copy(dst_hbm.at[pl.ds(base, C1)], didx)
            for i in range(C1 // L):
                sl = pl.ds(i * L, L)
                al = (plsc.load_gather(asrc_t, [sidx[sl]])
                      + plsc.load_gather(adst_t, [didx[sl]]))
                al = jnp.maximum(al, 0.0) + 0.2 * jnp.minimum(al, 0.0)
                ex_v[sl] = jnp.exp(al - mvec)
            pltpu.sync_copy(ex_v, ex_hbm.at[pl.ds(base, C1)])
            return 0
        lax.fori_loop(0, k1, body, 0)

    return pass1(a_src, a_dst, src_pad, dst_pad, m16)


def _sc_pass2(x, mix, ntp, k2):
    d = x.shape[1]
    rpt = ntp // NS

    @functools.partial(
        pl.kernel,
        out_type=jax.ShapeDtypeStruct((NC, ntp, DA), jnp.float32),
        mesh=_MESH,
        compiler_params=pltpu.CompilerParams(needs_layout_passes=False,
                                             use_tc_tiling_on_sc=False),
        scratch_types=[
            pltpu.VMEM((2, 3, C2), jnp.int32),
            pltpu.VMEM((2, C2, 128), jnp.float32),
            pltpu.VMEM((2, C2, DA), jnp.float32),
            pltpu.VMEM_SHARED((ntp, DA), jnp.float32),
            pltpu.SemaphoreType.DMA,
            pltpu.SemaphoreType.DMA,
            pltpu.SemaphoreType.DMA,
            pltpu.SemaphoreType.DMA,
        ],
    )
    def pass2(x_hbm, mix_hbm, acc_hbm,
              mix_v, xg, rows, acc_sh, g0, g1, s0, s1):
        ci = lax.axis_index("c")
        si = lax.axis_index("s")
        wid = ci * NS + si
        cbase = wid * k2
        gsem = (g0, g1)
        ssem = (s0, s1)

        for j in range(C2):
            for jj in range(DA // L):
                rows[0, j, pl.ds(jj * L, L)] = jnp.zeros((L,), jnp.float32)

        def zcopy(r, _):
            pltpu.sync_copy(rows.at[0],
                            acc_sh.at[pl.ds(si * rpt + r * C2, C2), :])
            return 0
        lax.fori_loop(0, rpt // C2, zcopy, 0)
        plsc.subcore_barrier()

        onehot = jnp.where(lax.iota(jnp.int32, L) == 0, 1.0, 0.0)
        ones = jnp.ones((L,), jnp.float32)

        def start_gather(p, c, drain):
            @pl.when(drain)
            def _():
                pltpu.make_async_copy(rows.at[p], acc_sh.at[mix_v.at[p, 1]],
                                      ssem[p]).wait()
            pltpu.sync_copy(mix_hbm.at[c], mix_v.at[p])
            pltpu.async_copy(x_hbm.at[mix_v.at[p, 0]], xg.at[p], gsem[p])

        def leg(p):
            pltpu.make_async_copy(x_hbm.at[mix_v.at[p, 0]], xg.at[p],
                                  gsem[p]).wait()
            for g in range(C2 // L):
                ev = plsc.bitcast(mix_v[p, 2, pl.ds(g * L, L)], jnp.float32)
                for lane in range(L):
                    r = g * L + lane
                    exb = ev[lane] * ones
                    for j in range(d // L):
                        sl = pl.ds(j * L, L)
                        rows[p, r, sl] = xg[p, r, sl] * exb
                    rows[p, r, pl.ds(d, L)] = exb * onehot
            pltpu.async_copy(rows.at[p], acc_sh.at[mix_v.at[p, 1]], ssem[p],
                             add=True)

        start_gather(0, cbase, jnp.bool_(False))

        def body(t, _):
            c = cbase + 2 * t
            start_gather(1, c + 1, t > 0)
            leg(0)

            @pl.when(t < k2 // 2 - 1)
            def _():
                start_gather(0, c + 2, jnp.bool_(True))
            leg(1)
            return 0
        lax.fori_loop(0, k2 // 2, body, 0)
        pltpu.make_async_copy(rows.at[0], acc_sh.at[mix_v.at[0, 1]], s0).wait()
        pltpu.make_async_copy(rows.at[1], acc_sh.at[mix_v.at[1, 1]], s1).wait()
        plsc.subcore_barrier()
        pltpu.sync_copy(acc_sh.at[pl.ds(si * rpt, rpt), :],
                        acc_hbm.at[ci, pl.ds(si * rpt, rpt), :])

    return pass2(x, mix)


def _final_body(nt, d, ap_ref, xt_ref, w_ref, b_ref, g_ref, bb_ref, out_ref):
    xagg = ap_ref[0, :nt, :d] + ap_ref[1, :nt, :d]
    den = ap_ref[0, :nt, d:d + 1] + ap_ref[1, :nt, d:d + 1]
    xn = xagg / jnp.maximum(den, 1e-16)
    has = (den > 0.0).astype(jnp.float32)
    h = jnp.dot(xn, w_ref[...], preferred_element_type=jnp.float32)
    h = h + has * b_ref[...][0][None, :]
    h = jnp.maximum(h, 0.0)
    t = h + xt_ref[...]
    mean = jnp.mean(t, axis=0, keepdims=True)
    var = jnp.mean((t - mean) ** 2, axis=0, keepdims=True)
    y = (t - mean) / jnp.sqrt(var + 1e-5) * g_ref[...][0][None, :] \
        + bb_ref[...][0][None, :]
    out_ref[...] = jnp.maximum(y, 0.0)


def kernel(x_ingredient, x_taste, edge_src, edge_dst, W_ing, b_ing, W_taste,
           b_taste, att_src, att_dst, Wk, bk, q, gamma, beta):
    n_ing, d = x_ingredient.shape
    nt = x_taste.shape[0]
    e = edge_src.shape[0]
    ntp = ((nt + 1 + NS * L - 1) // (NS * L)) * (NS * L)
    k1 = (e + NW * C1 - 1) // (NW * C1)
    ep = NW * C1 * k1
    k2 = ep // (NW * C2)

    a_src, ms = _matvec(x_ingredient, W_ing, att_src, b_ing, n_ing, BR)
    a_dst, md = _matvec(x_taste, W_taste, att_dst, b_taste, nt, BR)
    a_dst = a_dst[:ntp]
    t = ms + md
    m = jnp.maximum(t, 0.0) + 0.2 * jnp.minimum(t, 0.0)
    m16 = jnp.full((L,), 1.0, jnp.float32) * m

    src_pad = jnp.concatenate([edge_src, jnp.zeros((ep - e,), jnp.int32)])
    dst_pad = jnp.concatenate([edge_dst, jnp.full((ep - e,), nt, jnp.int32)])

    ex = _sc_pass1(a_src, a_dst, src_pad, dst_pad, m16, k1)
    nchunk = ep // C2
    mix = jnp.stack([src_pad.reshape(nchunk, C2),
                     dst_pad.reshape(nchunk, C2),
                     lax.bitcast_convert_type(ex, jnp.int32)
                        .reshape(nchunk, C2)], axis=1)
    acc_parts = _sc_pass2(x_ingredient, mix, ntp, k2)

    out_t = pl.pallas_call(
        functools.partial(_final_body, nt, d),
        out_shape=jax.ShapeDtypeStruct((nt, d), jnp.float32),
    )(acc_parts, x_taste, W_ing, b_ing[None, :], gamma[None, :], beta[None, :])
    return (x_ingredient, out_t)


# speedup vs baseline: 33.0745x; 1.0767x over previous
import functools

import jax
import jax.numpy as jnp
from jax import lax
from jax.experimental import pallas as pl
from jax.experimental.pallas import tpu as pltpu
from jax.experimental.pallas import tpu_sc as plsc

NC = 2
NS = 16
NW = NC * NS
L = 16
C1 = 256
C2 = 64
NB = 14
DA = 144
BR = 2048

_MESH = plsc.VectorSubcoreMesh(core_axis_name="c", subcore_axis_name="s")


def _matvec_body(nvalid, br, x_ref, w_ref, att_ref, b_ref, out_ref, max_ref):
    i = pl.program_id(0)
    u = jnp.dot(w_ref[...], att_ref[...][0][:, None],
                preferred_element_type=jnp.float32)
    c = jnp.sum(b_ref[...] * att_ref[...])
    s = jnp.dot(x_ref[...], u, preferred_element_type=jnp.float32) + c
    s2 = s.reshape(br // 128, 128)
    row = lax.broadcasted_iota(jnp.int32, (br // 128, 128), 0)
    col = lax.broadcasted_iota(jnp.int32, (br // 128, 128), 1)
    valid = (i * br + row * 128 + col) < nvalid
    out_ref[...] = jnp.where(valid, s2, 0.0)

    @pl.when(i == 0)
    def _():
        max_ref[0, 0] = -1e30

    max_ref[0, 0] = jnp.maximum(max_ref[0, 0],
                                jnp.max(jnp.where(valid, s2, -1e30)))


def _matvec(x, w, att, b, nvalid, br):
    n, d = x.shape
    nblk = (n + br - 1) // br
    out, mx = pl.pallas_call(
        functools.partial(_matvec_body, nvalid, br),
        grid=(nblk,),
        in_specs=[
            pl.BlockSpec((br, d), lambda i: (i, 0)),
            pl.BlockSpec((d, d), lambda i: (0, 0)),
            pl.BlockSpec((1, d), lambda i: (0, 0)),
            pl.BlockSpec((1, d), lambda i: (0, 0)),
        ],
        out_specs=[
            pl.BlockSpec((br // 128, 128), lambda i: (i, 0)),
            pl.BlockSpec(memory_space=pltpu.SMEM),
        ],
        out_shape=[
            jax.ShapeDtypeStruct((nblk * br // 128, 128), jnp.float32),
            jax.ShapeDtypeStruct((1, 1), jnp.float32),
        ],
    )(x, w, att[None, :], b[None, :])
    return out.reshape(-1), mx[0, 0]


def _sc_pass1(a_src, a_dst, src_pad, dst_pad, m16, k1):
    asp = a_src.shape[0]
    ntp = a_dst.shape[0]
    ep = src_pad.shape[0]

    @functools.partial(
        pl.kernel,
        out_type=jax.ShapeDtypeStruct((ep,), jnp.float32),
        mesh=_MESH,
        compiler_params=pltpu.CompilerParams(needs_layout_passes=False),
        scratch_types=[
            pltpu.VMEM((asp,), jnp.float32),
            pltpu.VMEM((ntp,), jnp.float32),
            pltpu.VMEM((C1,), jnp.int32),
            pltpu.VMEM((C1,), jnp.int32),
            pltpu.VMEM((C1,), jnp.float32),
            pltpu.VMEM((L,), jnp.float32),
        ],
    )
    def pass1(asrc_hbm, adst_hbm, src_hbm, dst_hbm, m_hbm, ex_hbm,
              asrc_t, adst_t, sidx, didx, ex_v, m_v):
        ci = lax.axis_index("c")
        si = lax.axis_index("s")
        wid = ci * NS + si
        pltpu.sync_copy(asrc_hbm, asrc_t)
        pltpu.sync_copy(adst_hbm, adst_t)
        pltpu.sync_copy(m_hbm, m_v)
        mvec = m_v[...]

        def body(k, _):
            base = wid * k1 * C1 + k * C1
            pltpu.sync_copy(src_hbm.at[pl.ds(base, C1)], sidx)
            pltpu.sync_copy(dst_hbm.at[pl.ds(base, C1)], didx)
            for i in range(C1 // L):
                sl = pl.ds(i * L, L)
                al = (plsc.load_gather(asrc_t, [sidx[sl]])
                      + plsc.load_gather(adst_t, [didx[sl]]))
                al = jnp.maximum(al, 0.0) + 0.2 * jnp.minimum(al, 0.0)
                ex_v[sl] = jnp.exp(al - mvec)
            pltpu.sync_copy(ex_v, ex_hbm.at[pl.ds(base, C1)])
            return 0
        lax.fori_loop(0, k1, body, 0)

    return pass1(a_src, a_dst, src_pad, dst_pad, m16)


def _sc_pass2(x, mix, ntp, k2):
    d = x.shape[1]
    rpt = ntp // NS

    @functools.partial(
        pl.kernel,
        out_type=jax.ShapeDtypeStruct((NC, ntp, DA), jnp.float32),
        mesh=_MESH,
        compiler_params=pltpu.CompilerParams(needs_layout_passes=False,
                                             use_tc_tiling_on_sc=False),
        scratch_types=[
            pltpu.VMEM((NB, 3, C2), jnp.int32),
            pltpu.VMEM((2, C2, 128), jnp.float32),
            pltpu.VMEM((2, C2, DA), jnp.float32),
            pltpu.VMEM_SHARED((ntp, DA), jnp.float32),
            pltpu.SemaphoreType.DMA,
            pltpu.SemaphoreType.DMA,
            pltpu.SemaphoreType.DMA,
            pltpu.SemaphoreType.DMA,
        ],
    )
    def pass2(x_hbm, mix_hbm, acc_hbm,
              mix_v, xg, rows, acc_sh, g0, g1, s0, s1):
        ci = lax.axis_index("c")
        si = lax.axis_index("s")
        wid = ci * NS + si
        cbase = wid * k2
        ns = k2 // NB
        gsem = (g0, g1)
        ssem = (s0, s1)

        for j in range(C2):
            for jj in range(DA // L):
                rows[0, j, pl.ds(jj * L, L)] = jnp.zeros((L,), jnp.float32)

        def zcopy(r, _):
            pltpu.sync_copy(rows.at[0],
                            acc_sh.at[pl.ds(si * rpt + r * C2, C2), :])
            return 0
        lax.fori_loop(0, rpt // C2, zcopy, 0)
        plsc.subcore_barrier()

        onehot = jnp.where(lax.iota(jnp.int32, L) == 0, 1.0, 0.0)
        ones = jnp.ones((L,), jnp.float32)

        def start_gather(p, cj):
            pltpu.async_copy(x_hbm.at[mix_v.at[cj, 0]], xg.at[p], gsem[p])

        def drain_scatter(p):
            pltpu.make_async_copy(rows.at[p], acc_sh.at[mix_v.at[0, 1]],
                                  ssem[p]).wait()

        def leg(p, t, cj):
            pltpu.make_async_copy(x_hbm.at[mix_v.at[cj, 0]], xg.at[p],
                                  gsem[p]).wait()

            @pl.when(t > 0)
            def _():
                drain_scatter(p)
            for g in range(C2 // L):
                ev = plsc.bitcast(mix_v[cj, 2, pl.ds(g * L, L)], jnp.float32)
                for lane in range(L):
                    r = g * L + lane
                    exb = ev[lane] * ones
                    for j in range(d // L):
                        sl = pl.ds(j * L, L)
                        rows[p, r, sl] = xg[p, r, sl] * exb
                    rows[p, r, pl.ds(d, L)] = exb * onehot
            pltpu.async_copy(rows.at[p], acc_sh.at[mix_v.at[cj, 1]], ssem[p],
                             add=True)

        def super_body(s, _):
            @pl.when(s > 0)
            def _():
                drain_scatter(0)
                drain_scatter(1)
            pltpu.sync_copy(mix_hbm.at[pl.ds(cbase + s * NB, NB)], mix_v)
            start_gather(0, 0)

            def body(t, _):
                start_gather(1, 2 * t + 1)
                leg(0, t, 2 * t)

                @pl.when(t < NB // 2 - 1)
                def _():
                    start_gather(0, 2 * t + 2)
                leg(1, t, 2 * t + 1)
                return 0
            lax.fori_loop(0, NB // 2, body, 0)
            return 0
        lax.fori_loop(0, ns, super_body, 0)
        drain_scatter(0)
        drain_scatter(1)
        plsc.subcore_barrier()
        pltpu.sync_copy(acc_sh.at[pl.ds(si * rpt, rpt), :],
                        acc_hbm.at[ci, pl.ds(si * rpt, rpt), :])

    return pass2(x, mix)


def _final_body(nt, d, ap_ref, xt_ref, w_ref, b_ref, g_ref, bb_ref, out_ref):
    xagg = ap_ref[0, :nt, :d] + ap_ref[1, :nt, :d]
    den = ap_ref[0, :nt, d:d + 1] + ap_ref[1, :nt, d:d + 1]
    xn = xagg / jnp.maximum(den, 1e-16)
    has = (den > 0.0).astype(jnp.float32)
    h = jnp.dot(xn, w_ref[...], preferred_element_type=jnp.float32)
    h = h + has * b_ref[...][0][None, :]
    h = jnp.maximum(h, 0.0)
    t = h + xt_ref[...]
    mean = jnp.mean(t, axis=0, keepdims=True)
    var = jnp.mean((t - mean) ** 2, axis=0, keepdims=True)
    y = (t - mean) / jnp.sqrt(var + 1e-5) * g_ref[...][0][None, :] \
        + bb_ref[...][0][None, :]
    out_ref[...] = jnp.maximum(y, 0.0)


def kernel(x_ingredient, x_taste, edge_src, edge_dst, W_ing, b_ing, W_taste,
           b_taste, att_src, att_dst, Wk, bk, q, gamma, beta):
    n_ing, d = x_ingredient.shape
    nt = x_taste.shape[0]
    e = edge_src.shape[0]
    ntp = ((nt + 1 + NS * L - 1) // (NS * L)) * (NS * L)
    k1 = (e + NW * C1 - 1) // (NW * C1)
    ep = NW * C1 * k1
    k2 = ep // (NW * C2)

    a_src, ms = _matvec(x_ingredient, W_ing, att_src, b_ing, n_ing, BR)
    a_dst, md = _matvec(x_taste, W_taste, att_dst, b_taste, nt, BR)
    a_dst = a_dst[:ntp]
    t = ms + md
    m = jnp.maximum(t, 0.0) + 0.2 * jnp.minimum(t, 0.0)
    m16 = jnp.full((L,), 1.0, jnp.float32) * m

    src_pad = jnp.concatenate([edge_src, jnp.zeros((ep - e,), jnp.int32)])
    dst_pad = jnp.concatenate([edge_dst, jnp.full((ep - e,), nt, jnp.int32)])

    ex = _sc_pass1(a_src, a_dst, src_pad, dst_pad, m16, k1)
    nchunk = ep // C2
    mix = jnp.stack([src_pad.reshape(nchunk, C2),
                     dst_pad.reshape(nchunk, C2),
                     lax.bitcast_convert_type(ex, jnp.int32)
                        .reshape(nchunk, C2)], axis=1)
    acc_parts = _sc_pass2(x_ingredient, mix, ntp, k2)

    out_t = pl.pallas_call(
        functools.partial(_final_body, nt, d),
        out_shape=jax.ShapeDtypeStruct((nt, d), jnp.float32),
    )(acc_parts, x_taste, W_ing, b_ing[None, :], gamma[None, :], beta[None, :])
    return (x_ingredient, out_t)
